# baseline (speedup 1.0000x reference)
import functools

import jax
import jax.numpy as jnp
from jax import lax
from jax.experimental import pallas as pl
from jax.experimental.pallas import tpu as pltpu
from jax.experimental.pallas import tpu_sc as plsc

N = 2048
DIM = 1024
H = 16
HD = 64
KNN = 64
SCALE = 1.0 / (HD ** 0.5)
LOG2E = 1.4426950408889634
QSCALE = SCALE * LOG2E
BQ = 512
NQB = N // BQ

_CONTRACT_LAST = (((1,), (1,)), ((), ()))
_CONTRACT_STD = (((1,), (0,)), ((), ()))

_NC = 2
_NS = 16
_LANES = 16
_NW = _NC * _NS
_ROWS_W = N // _NW
_RC = 32
_NCHUNK = _ROWS_W // _RC

def _sc_counts_kernel_body(routes_hbm, out_hbm, routes_v, buf_v):
    wid = lax.axis_index("s") * _NC + lax.axis_index("c")
    base_row = wid * _ROWS_W
    pltpu.sync_copy(routes_hbm.at[pl.ds(base_row * KNN, _ROWS_W * KNN)],
                    routes_v)
    ones = jnp.full((_LANES,), 1.0, jnp.float32)
    zeros = jnp.zeros((_LANES,), jnp.float32)
    lane = lax.iota(jnp.int32, _LANES)
    is0 = lane == 0
    for chunk in range(_NCHUNK):
        @plsc.parallel_loop(0, _RC * N // _LANES, 1, unroll=8)
        def _zero(i):
            buf_v[pl.ds(i * _LANES, _LANES)] = zeros
        for r in range(_RC):
            rr = chunk * _RC + r
            for kc in range(KNN // _LANES):
                cvec = routes_v[pl.ds(rr * KNN + kc * _LANES, _LANES)]
                if kc == 0:
                    flat = jnp.where(is0, _RC * N, r * N + cvec)
                    plsc.addupdate_scatter(buf_v, [flat], ones)
                else:
                    plsc.addupdate_scatter(buf_v, [r * N + cvec], ones)
        for g in range(_RC // _LANES):
            gidx = (chunk * _RC + g * _LANES + lane) * KNN
            cvec0 = plsc.load_gather(routes_v, [gidx])
            flat0 = (g * _LANES + lane) * N + cvec0
            plsc.addupdate_scatter(buf_v, [flat0], ones)
        pltpu.sync_copy(buf_v.at[pl.ds(0, _RC * N)],
                        out_hbm.at[pl.ds((base_row + chunk * _RC) * N,
                                         _RC * N)])


_sc_counts_kernel = None


def _build_counts(routes):
    global _sc_counts_kernel
    if _sc_counts_kernel is None:
        _sc_counts_kernel = pl.kernel(
            _sc_counts_kernel_body,
            mesh=plsc.VectorSubcoreMesh(core_axis_name="c",
                                        subcore_axis_name="s"),
            out_type=jax.ShapeDtypeStruct((N * N,), jnp.float32),
            scratch_types=[
                pltpu.VMEM((_ROWS_W * KNN,), jnp.int32),
                pltpu.VMEM((_RC * N + _LANES,), jnp.float32),
            ],
            compiler_params=pltpu.CompilerParams(needs_layout_passes=False),
        )
    return _sc_counts_kernel(routes.reshape(N * KNN)).reshape(N, N)


def _qkv_kernel(x_ref, w_ref, b_ref, q_ref, k_ref, v_ref):
    xb = x_ref[...].astype(jnp.bfloat16)
    wb = w_ref[...].astype(jnp.bfloat16)
    acc = lax.dot_general(xb, wb, _CONTRACT_LAST,
                          preferred_element_type=jnp.float32)
    acc = acc + b_ref[0:1, :]
    q_ref[...] = (acc[:, :DIM] * QSCALE).astype(jnp.bfloat16)
    k_ref[...] = acc[:, DIM:2 * DIM].astype(jnp.bfloat16)
    v_ref[...] = acc[:, 2 * DIM:].astype(jnp.bfloat16)


def _qkv_proj(x, Wqkv, bqkv):
    shp = jax.ShapeDtypeStruct((N, DIM), jnp.bfloat16)
    return pl.pallas_call(
        _qkv_kernel,
        in_specs=[
            pl.BlockSpec((N, DIM), lambda: (0, 0)),
            pl.BlockSpec((3 * DIM, DIM), lambda: (0, 0)),
            pl.BlockSpec((1, 3 * DIM), lambda: (0, 0)),
        ],
        out_specs=(pl.BlockSpec((N, DIM), lambda: (0, 0)),) * 3,
        out_shape=(shp, shp, shp),
    )(x, Wqkv, bqkv.reshape(1, 3 * DIM))


def _attn_kernel(q_ref, k_ref, v_ref, c_ref, wo_ref, bo_ref, o_ref):
    c = c_ref[...]
    ones_col = jnp.ones((N, 1), jnp.bfloat16)
    y = []
    for h in range(H):
        q = q_ref[:, h * HD:(h + 1) * HD]
        k = k_ref[:, h * HD:(h + 1) * HD]
        v = v_ref[:, h * HD:(h + 1) * HD]
        s = lax.dot_general(q, k, _CONTRACT_LAST,
                            preferred_element_type=jnp.float32)
        p = (c * jnp.exp2(s)).astype(jnp.bfloat16)
        v_ext = jnp.concatenate([v, ones_col], axis=1)
        o_ext = lax.dot_general(p, v_ext, _CONTRACT_STD,
                                preferred_element_type=jnp.float32)
        o = o_ext[:, :HD] / o_ext[:, HD:HD + 1]
        y.append(o.astype(jnp.bfloat16))
    yb = jnp.concatenate(y, axis=1)
    wo = wo_ref[...].astype(jnp.bfloat16)
    acc = lax.dot_general(yb, wo, _CONTRACT_LAST,
                          preferred_element_type=jnp.float32)
    o_ref[...] = acc + bo_ref[0:1, :]


def _attention_out(qm, km, vm, counts, Wout, bout):
    return pl.pallas_call(
        _attn_kernel,
        grid=(NQB,),
        in_specs=[
            pl.BlockSpec((BQ, DIM), lambda i: (i, 0)),
            pl.BlockSpec((N, DIM), lambda i: (0, 0)),
            pl.BlockSpec((N, DIM), lambda i: (0, 0)),
            pl.BlockSpec((BQ, N), lambda i: (i, 0)),
            pl.BlockSpec((DIM, DIM), lambda i: (0, 0)),
            pl.BlockSpec((1, DIM), lambda i: (0, 0)),
        ],
        out_specs=pl.BlockSpec((BQ, DIM), lambda i: (i, 0)),
        out_shape=jax.ShapeDtypeStruct((N, DIM), jnp.float32),
    )(qm, km, vm, counts, Wout, bout.reshape(1, DIM))


def kernel(x, Wqkv, bqkv, Wout, bout, routes):
    qm, km, vm = _qkv_proj(x.reshape(N, DIM), Wqkv, bqkv)
    counts = _build_counts(routes)
    out = _attention_out(qm, km, vm, counts, Wout, bout)
    return out.reshape(1, N, DIM)


# speedup vs baseline: 1.0250x; 1.0250x over previous
import functools

import jax
import jax.numpy as jnp
from jax import lax
from jax.experimental import pallas as pl
from jax.experimental.pallas import tpu as pltpu
from jax.experimental.pallas import tpu_sc as plsc

N = 2048
DIM = 1024
H = 16
HD = 64
KNN = 64
SCALE = 1.0 / (HD ** 0.5)
LOG2E = 1.4426950408889634
QSCALE = SCALE * LOG2E
BQ = 512
NQB = N // BQ

_CONTRACT_LAST = (((1,), (1,)), ((), ()))
_CONTRACT_STD = (((1,), (0,)), ((), ()))

_NC = 2
_NS = 16
_LANES = 16
_NW = _NC * _NS
_ROWS_W = N // _NW
_RC = 32
_NCHUNK = _ROWS_W // _RC

def _sc_counts_kernel_body(routes_hbm, out_hbm, routes_v, buf_v):
    wid = lax.axis_index("s") * _NC + lax.axis_index("c")
    base_row = wid * _ROWS_W
    pltpu.sync_copy(routes_hbm.at[pl.ds(base_row * KNN, _ROWS_W * KNN)],
                    routes_v)
    ones = jnp.full((_LANES,), 1.0, jnp.float32)
    zeros = jnp.zeros((_LANES,), jnp.float32)
    lane = lax.iota(jnp.int32, _LANES)
    is0 = lane == 0
    for chunk in range(_NCHUNK):
        @plsc.parallel_loop(0, _RC * N // _LANES, 1, unroll=8)
        def _zero(i):
            buf_v[pl.ds(i * _LANES, _LANES)] = zeros
        for r in range(_RC):
            rr = chunk * _RC + r
            for kc in range(KNN // _LANES):
                cvec = routes_v[pl.ds(rr * KNN + kc * _LANES, _LANES)]
                if kc == 0:
                    flat = jnp.where(is0, _RC * N, r * N + cvec)
                    plsc.addupdate_scatter(buf_v, [flat], ones)
                else:
                    plsc.addupdate_scatter(buf_v, [r * N + cvec], ones)
        for g in range(_RC // _LANES):
            gidx = (chunk * _RC + g * _LANES + lane) * KNN
            cvec0 = plsc.load_gather(routes_v, [gidx])
            flat0 = (g * _LANES + lane) * N + cvec0
            plsc.addupdate_scatter(buf_v, [flat0], ones)
        pltpu.sync_copy(buf_v.at[pl.ds(0, _RC * N)],
                        out_hbm.at[pl.ds((base_row + chunk * _RC) * N,
                                         _RC * N)])


_sc_counts_kernel = None


def _build_counts(routes):
    global _sc_counts_kernel
    if _sc_counts_kernel is None:
        _sc_counts_kernel = pl.kernel(
            _sc_counts_kernel_body,
            mesh=plsc.VectorSubcoreMesh(core_axis_name="c",
                                        subcore_axis_name="s"),
            out_type=jax.ShapeDtypeStruct((N * N,), jnp.float32),
            scratch_types=[
                pltpu.VMEM((_ROWS_W * KNN,), jnp.int32),
                pltpu.VMEM((_RC * N + _LANES,), jnp.float32),
            ],
            compiler_params=pltpu.CompilerParams(needs_layout_passes=False),
        )
    return _sc_counts_kernel(routes.reshape(N * KNN)).reshape(N, N)


def _qkv_kernel(x_ref, w_ref, b_ref, q_ref, k_ref, v_ref):
    xb = x_ref[...].astype(jnp.bfloat16)
    wb = w_ref[...].astype(jnp.bfloat16)
    acc = lax.dot_general(xb, wb, _CONTRACT_LAST,
                          preferred_element_type=jnp.float32)
    acc = acc + b_ref[0:1, :]
    q_ref[...] = (acc[:, :DIM] * QSCALE).astype(jnp.bfloat16)
    k_ref[...] = acc[:, DIM:2 * DIM].astype(jnp.bfloat16)
    v_ref[...] = acc[:, 2 * DIM:].astype(jnp.bfloat16)


def _qkv_proj(x, Wqkv, bqkv):
    shp = jax.ShapeDtypeStruct((N, DIM), jnp.bfloat16)
    return pl.pallas_call(
        _qkv_kernel,
        in_specs=[
            pl.BlockSpec((N, DIM), lambda: (0, 0)),
            pl.BlockSpec((3 * DIM, DIM), lambda: (0, 0)),
            pl.BlockSpec((1, 3 * DIM), lambda: (0, 0)),
        ],
        out_specs=(pl.BlockSpec((N, DIM), lambda: (0, 0)),) * 3,
        out_shape=(shp, shp, shp),
    )(x, Wqkv, bqkv.reshape(1, 3 * DIM))


def _attn_kernel(q_ref, k_ref, v_ref, c_ref, wo_ref, bo_ref, o_ref):
    c = c_ref[...].astype(jnp.bfloat16)
    ones_col = jnp.ones((N, 1), jnp.bfloat16)
    y = []
    for h in range(H):
        q = q_ref[:, h * HD:(h + 1) * HD]
        k = k_ref[:, h * HD:(h + 1) * HD]
        v = v_ref[:, h * HD:(h + 1) * HD]
        s = lax.dot_general(q, k, _CONTRACT_LAST,
                            preferred_element_type=jnp.float32)
        p = c * jnp.exp2(s).astype(jnp.bfloat16)
        v_ext = jnp.concatenate([v, ones_col], axis=1)
        o_ext = lax.dot_general(p, v_ext, _CONTRACT_STD,
                                preferred_element_type=jnp.float32)
        o = o_ext[:, :HD] / o_ext[:, HD:HD + 1]
        y.append(o.astype(jnp.bfloat16))
    yb = jnp.concatenate(y, axis=1)
    wo = wo_ref[...].astype(jnp.bfloat16)
    acc = lax.dot_general(yb, wo, _CONTRACT_LAST,
                          preferred_element_type=jnp.float32)
    o_ref[...] = acc + bo_ref[0:1, :]


def _attention_out(qm, km, vm, counts, Wout, bout):
    return pl.pallas_call(
        _attn_kernel,
        grid=(NQB,),
        in_specs=[
            pl.BlockSpec((BQ, DIM), lambda i: (i, 0)),
            pl.BlockSpec((N, DIM), lambda i: (0, 0)),
            pl.BlockSpec((N, DIM), lambda i: (0, 0)),
            pl.BlockSpec((BQ, N), lambda i: (i, 0)),
            pl.BlockSpec((DIM, DIM), lambda i: (0, 0)),
            pl.BlockSpec((1, DIM), lambda i: (0, 0)),
        ],
        out_specs=pl.BlockSpec((BQ, DIM), lambda i: (i, 0)),
        out_shape=jax.ShapeDtypeStruct((N, DIM), jnp.float32),
    )(qm, km, vm, counts, Wout, bout.reshape(1, DIM))


def kernel(x, Wqkv, bqkv, Wout, bout, routes):
    qm, km, vm = _qkv_proj(x.reshape(N, DIM), Wqkv, bqkv)
    counts = _build_counts(routes)
    out = _attention_out(qm, km, vm, counts, Wout, bout)
    return out.reshape(1, N, DIM)
